# Initial kernel scaffold; baseline (speedup 1.0000x reference)
import jax
import jax.numpy as jnp
from jax.experimental import pallas as pl


def kernel(vertices, faces, face_probs):
    raise NotImplementedError("write your pallas kernel here")


# speedup vs baseline: 10.5486x; 10.5486x over previous
import jax
import jax.numpy as jnp
from jax.experimental import pallas as pl

_F = 2000
_E = 3 * _F
_EPAD = 6144
_BI = 256
_BJ = 1024
_DIST2 = (1.0 + 1e-6) ** 2
_CROSS2 = (1e-5 - 1e-8) ** 2
_FPAD = 2048


def _pair_kernel(rdat_ref, cdat_ref, out_ref):
    j = pl.program_id(1)
    r = rdat_ref[...]
    c = cdat_ref[...]
    ddx = r[:, 0:1] - c[0:1, :]
    ddy = r[:, 1:2] - c[1:2, :]
    ddz = r[:, 2:3] - c[2:3, :]
    s = ddx * ddx + ddy * ddy + ddz * ddz
    c0 = r[:, 4:5] * c[5:6, :] - r[:, 5:6] * c[4:5, :]
    c1 = r[:, 5:6] * c[3:4, :] - r[:, 3:4] * c[5:6, :]
    c2 = r[:, 3:4] * c[4:5, :] - r[:, 4:5] * c[3:4, :]
    g = c0 * c0 + c1 * c1 + c2 * c2
    m = (s < _DIST2) & (g > _CROSS2)
    part = jnp.sum(m.astype(jnp.float32), axis=1, keepdims=True)

    @pl.when(j == 0)
    def _init():
        out_ref[...] = part

    @pl.when(j > 0)
    def _acc():
        out_ref[...] += part


def _loss_kernel(n0_ref, n1_ref, n2_ref, fp_ref, out_ref):
    cc = n0_ref[...] + n1_ref[...] + n2_ref[...]
    cc = jnp.clip(cc, 0.0, 100.0)
    out_ref[...] = (jnp.sum(cc * fp_ref[...]) / _F)[None, None]


def kernel(vertices, faces, face_probs):
    f0 = faces[:, 0]
    f1 = faces[:, 1]
    f2 = faces[:, 2]
    starts = jnp.concatenate([f0, f1, f2])
    ends = jnp.concatenate([f1, f2, f0])
    p0 = vertices[starts]
    p1 = vertices[ends]
    cen = (p0 + p1) * 0.5
    d = (p1 - p0) + 1e-8
    rdat = jnp.concatenate([cen, d], axis=1)
    rdat = jnp.pad(rdat, ((0, _EPAD - _E), (0, 0)))
    rdat = rdat.at[_E:, 0].set(1e6)
    cdat = rdat.T

    nsum = pl.pallas_call(
        _pair_kernel,
        grid=(_EPAD // _BI, _EPAD // _BJ),
        in_specs=[
            pl.BlockSpec((_BI, 6), lambda i, j: (i, 0)),
            pl.BlockSpec((6, _BJ), lambda i, j: (0, j)),
        ],
        out_specs=pl.BlockSpec((_BI, 1), lambda i, j: (i, 0)),
        out_shape=jax.ShapeDtypeStruct((_EPAD, 1), jnp.float32),
    )(rdat, cdat)

    n = nsum[:_E, 0]
    n0 = jnp.pad(n[0::3], (0, _FPAD - _F))[None, :]
    n1 = jnp.pad(n[1::3], (0, _FPAD - _F))[None, :]
    n2 = jnp.pad(n[2::3], (0, _FPAD - _F))[None, :]
    fp = jnp.pad(face_probs, (0, _FPAD - _F))[None, :]

    loss = pl.pallas_call(
        _loss_kernel,
        out_shape=jax.ShapeDtypeStruct((1, 1), jnp.float32),
    )(n0, n1, n2, fp)
    return loss[0, 0]


# speedup vs baseline: 14.2667x; 1.3525x over previous
import jax
import jax.numpy as jnp
from jax.experimental import pallas as pl

_F = 2000
_E = 3 * _F
_EPAD = 6144
_BI = 256
_BJ = 1024
_DIST2 = (1.0 + 1e-6) ** 2
_CROSS2 = (1e-5 - 1e-8) ** 2
_FPAD = 2048


def _pair_kernel(rdat_ref, cdat_ref, out_ref):
    j = pl.program_id(1)
    r = rdat_ref[...]
    c = cdat_ref[...]
    a2 = jnp.dot(r[:, 0:3], c[0:3, :], preferred_element_type=jnp.float32)
    gd = jnp.dot(r[:, 3:6], c[3:6, :], preferred_element_type=jnp.float32)
    s = (r[:, 6:7] + c[6:7, :]) + a2
    g = r[:, 7:8] * c[7:8, :] - gd * gd
    m = (s < _DIST2) & (g > _CROSS2)
    part = jnp.sum(m.astype(jnp.float32), axis=1, keepdims=True)

    @pl.when(j == 0)
    def _init():
        out_ref[...] = part

    @pl.when(j > 0)
    def _acc():
        out_ref[...] += part


def _loss_kernel(n0_ref, n1_ref, n2_ref, fp_ref, out_ref):
    cc = n0_ref[...] + n1_ref[...] + n2_ref[...]
    cc = jnp.clip(cc, 0.0, 100.0)
    out_ref[...] = (jnp.sum(cc * fp_ref[...]) / _F)[None, None]


def kernel(vertices, faces, face_probs):
    f0 = faces[:, 0]
    f1 = faces[:, 1]
    f2 = faces[:, 2]
    starts = jnp.concatenate([f0, f1, f2])
    ends = jnp.concatenate([f1, f2, f0])
    p0 = vertices[starts]
    p1 = vertices[ends]
    cen = (p0 + p1) * 0.5
    d = (p1 - p0) + 1e-8
    nc = jnp.sum(cen * cen, axis=1, keepdims=True)
    nd = jnp.sum(d * d, axis=1, keepdims=True)
    rdat = jnp.concatenate([cen, d, nc, nd], axis=1)
    rdat = jnp.pad(rdat, ((0, _EPAD - _E), (0, 0)))
    rdat = rdat.at[_E:, 6].set(1e12)
    cdat = jnp.concatenate(
        [-2.0 * cen, d, nc, nd], axis=1)
    cdat = jnp.pad(cdat, ((0, _EPAD - _E), (0, 0)))
    cdat = cdat.at[_E:, 6].set(1e12)
    cdat = cdat.T

    nsum = pl.pallas_call(
        _pair_kernel,
        grid=(_EPAD // _BI, _EPAD // _BJ),
        in_specs=[
            pl.BlockSpec((_BI, 8), lambda i, j: (i, 0)),
            pl.BlockSpec((8, _BJ), lambda i, j: (0, j)),
        ],
        out_specs=pl.BlockSpec((_BI, 1), lambda i, j: (i, 0)),
        out_shape=jax.ShapeDtypeStruct((_EPAD, 1), jnp.float32),
    )(rdat, cdat)

    n = nsum[:_E, 0]
    n0 = jnp.pad(n[0::3], (0, _FPAD - _F))[None, :]
    n1 = jnp.pad(n[1::3], (0, _FPAD - _F))[None, :]
    n2 = jnp.pad(n[2::3], (0, _FPAD - _F))[None, :]
    fp = jnp.pad(face_probs, (0, _FPAD - _F))[None, :]

    loss = pl.pallas_call(
        _loss_kernel,
        out_shape=jax.ShapeDtypeStruct((1, 1), jnp.float32),
    )(n0, n1, n2, fp)
    return loss[0, 0]


# speedup vs baseline: 19.1274x; 1.3407x over previous
import jax
import jax.numpy as jnp
from jax.experimental import pallas as pl

_F = 2000
_E = 3 * _F
_EPAD = 6144
_BI = 512
_BJ = 2048
_DIST2 = (1.0 + 1e-6) ** 2
_CROSS2 = (1e-5 - 1e-8) ** 2
_FPAD = 2048


def _pair_kernel(rdat_ref, cdat_ref, out_ref):
    j = pl.program_id(1)
    r = rdat_ref[...]
    c = cdat_ref[...]
    a2 = jnp.dot(r[:, 0:3], c[0:3, :], preferred_element_type=jnp.float32)
    gd = jnp.dot(r[:, 3:6], c[3:6, :], preferred_element_type=jnp.float32)
    s = (r[:, 6:7] + c[6:7, :]) + a2
    g = r[:, 7:8] * c[7:8, :] - gd * gd
    m = (s < _DIST2) & (g > _CROSS2)
    part = jnp.sum(m.astype(jnp.float32), axis=1, keepdims=True)

    @pl.when(j == 0)
    def _init():
        out_ref[...] = part

    @pl.when(j > 0)
    def _acc():
        out_ref[...] += part


def _loss_kernel(n0_ref, n1_ref, n2_ref, fp_ref, out_ref):
    cc = n0_ref[...] + n1_ref[...] + n2_ref[...]
    cc = jnp.clip(cc, 0.0, 100.0)
    out_ref[...] = (jnp.sum(cc * fp_ref[...]) / _F)[None, None]


def kernel(vertices, faces, face_probs):
    f0 = faces[:, 0]
    f1 = faces[:, 1]
    f2 = faces[:, 2]
    starts = jnp.concatenate([f0, f1, f2])
    ends = jnp.concatenate([f1, f2, f0])
    p0 = vertices[starts]
    p1 = vertices[ends]
    cen = (p0 + p1) * 0.5
    d = (p1 - p0) + 1e-8
    nc = jnp.sum(cen * cen, axis=1, keepdims=True)
    nd = jnp.sum(d * d, axis=1, keepdims=True)
    rdat = jnp.concatenate([cen, d, nc, nd], axis=1)
    rdat = jnp.pad(rdat, ((0, _EPAD - _E), (0, 0)))
    rdat = rdat.at[_E:, 6].set(1e12)
    cdat = jnp.concatenate(
        [-2.0 * cen, d, nc, nd], axis=1)
    cdat = jnp.pad(cdat, ((0, _EPAD - _E), (0, 0)))
    cdat = cdat.at[_E:, 6].set(1e12)
    cdat = cdat.T

    nsum = pl.pallas_call(
        _pair_kernel,
        grid=(_EPAD // _BI, _EPAD // _BJ),
        in_specs=[
            pl.BlockSpec((_BI, 8), lambda i, j: (i, 0)),
            pl.BlockSpec((8, _BJ), lambda i, j: (0, j)),
        ],
        out_specs=pl.BlockSpec((_BI, 1), lambda i, j: (i, 0)),
        out_shape=jax.ShapeDtypeStruct((_EPAD, 1), jnp.float32),
    )(rdat, cdat)

    n = nsum[:_E, 0]
    n0 = jnp.pad(n[0::3], (0, _FPAD - _F))[None, :]
    n1 = jnp.pad(n[1::3], (0, _FPAD - _F))[None, :]
    n2 = jnp.pad(n[2::3], (0, _FPAD - _F))[None, :]
    fp = jnp.pad(face_probs, (0, _FPAD - _F))[None, :]

    loss = pl.pallas_call(
        _loss_kernel,
        out_shape=jax.ShapeDtypeStruct((1, 1), jnp.float32),
    )(n0, n1, n2, fp)
    return loss[0, 0]


# speedup vs baseline: 19.5596x; 1.0226x over previous
import jax
import jax.numpy as jnp
from jax.experimental import pallas as pl

_F = 2000
_E = 3 * _F
_EPAD = 6144
_BI = 512
_BJ = 3072
_DIST2 = (1.0 + 1e-6) ** 2
_CROSS2 = (1e-5 - 1e-8) ** 2
_FPAD = 2048


def _pair_kernel(rdat_ref, cdat_ref, out_ref):
    j = pl.program_id(1)
    r = rdat_ref[...]
    c = cdat_ref[...]
    a2 = jnp.dot(r[:, 0:3], c[0:3, :], preferred_element_type=jnp.float32)
    gd = jnp.dot(r[:, 3:6], c[3:6, :], preferred_element_type=jnp.float32)
    s = (r[:, 6:7] + c[6:7, :]) + a2
    g = r[:, 7:8] * c[7:8, :] - gd * gd
    m = (s < _DIST2) & (g > _CROSS2)
    part = jnp.sum(m.astype(jnp.float32), axis=1, keepdims=True)

    @pl.when(j == 0)
    def _init():
        out_ref[...] = part

    @pl.when(j > 0)
    def _acc():
        out_ref[...] += part


def _loss_kernel(n0_ref, n1_ref, n2_ref, fp_ref, out_ref):
    cc = n0_ref[...] + n1_ref[...] + n2_ref[...]
    cc = jnp.clip(cc, 0.0, 100.0)
    out_ref[...] = (jnp.sum(cc * fp_ref[...]) / _F)[None, None]


def kernel(vertices, faces, face_probs):
    f0 = faces[:, 0]
    f1 = faces[:, 1]
    f2 = faces[:, 2]
    starts = jnp.concatenate([f0, f1, f2])
    ends = jnp.concatenate([f1, f2, f0])
    p0 = vertices[starts]
    p1 = vertices[ends]
    cen = (p0 + p1) * 0.5
    d = (p1 - p0) + 1e-8
    nc = jnp.sum(cen * cen, axis=1, keepdims=True)
    nd = jnp.sum(d * d, axis=1, keepdims=True)
    rdat = jnp.concatenate([cen, d, nc, nd], axis=1)
    rdat = jnp.pad(rdat, ((0, _EPAD - _E), (0, 0)))
    rdat = rdat.at[_E:, 6].set(1e12)
    cdat = jnp.concatenate(
        [-2.0 * cen, d, nc, nd], axis=1)
    cdat = jnp.pad(cdat, ((0, _EPAD - _E), (0, 0)))
    cdat = cdat.at[_E:, 6].set(1e12)
    cdat = cdat.T

    nsum = pl.pallas_call(
        _pair_kernel,
        grid=(_EPAD // _BI, _EPAD // _BJ),
        in_specs=[
            pl.BlockSpec((_BI, 8), lambda i, j: (i, 0)),
            pl.BlockSpec((8, _BJ), lambda i, j: (0, j)),
        ],
        out_specs=pl.BlockSpec((_BI, 1), lambda i, j: (i, 0)),
        out_shape=jax.ShapeDtypeStruct((_EPAD, 1), jnp.float32),
    )(rdat, cdat)

    n = nsum[:_E, 0]
    n0 = jnp.pad(n[0::3], (0, _FPAD - _F))[None, :]
    n1 = jnp.pad(n[1::3], (0, _FPAD - _F))[None, :]
    n2 = jnp.pad(n[2::3], (0, _FPAD - _F))[None, :]
    fp = jnp.pad(face_probs, (0, _FPAD - _F))[None, :]

    loss = pl.pallas_call(
        _loss_kernel,
        out_shape=jax.ShapeDtypeStruct((1, 1), jnp.float32),
    )(n0, n1, n2, fp)
    return loss[0, 0]


# speedup vs baseline: 19.9065x; 1.0177x over previous
import jax
import jax.numpy as jnp
from jax.experimental import pallas as pl

_F = 2000
_E = 3 * _F
_EPAD = 6144
_BI = 1024
_BJ = 2048
_DIST2 = (1.0 + 1e-6) ** 2
_CROSS2 = (1e-5 - 1e-8) ** 2
_FPAD = 2048


def _pair_kernel(rdat_ref, cdat_ref, out_ref):
    j = pl.program_id(1)
    r = rdat_ref[...]
    c = cdat_ref[...]
    a2 = jnp.dot(r[:, 0:3], c[0:3, :], preferred_element_type=jnp.float32)
    gd = jnp.dot(r[:, 3:6], c[3:6, :], preferred_element_type=jnp.float32)
    s = (r[:, 6:7] + c[6:7, :]) + a2
    g = r[:, 7:8] * c[7:8, :] - gd * gd
    m = (s < _DIST2) & (g > _CROSS2)
    part = jnp.sum(m.astype(jnp.float32), axis=1, keepdims=True)

    @pl.when(j == 0)
    def _init():
        out_ref[...] = part

    @pl.when(j > 0)
    def _acc():
        out_ref[...] += part


def _loss_kernel(n0_ref, n1_ref, n2_ref, fp_ref, out_ref):
    cc = n0_ref[...] + n1_ref[...] + n2_ref[...]
    cc = jnp.clip(cc, 0.0, 100.0)
    out_ref[...] = (jnp.sum(cc * fp_ref[...]) / _F)[None, None]


def kernel(vertices, faces, face_probs):
    f0 = faces[:, 0]
    f1 = faces[:, 1]
    f2 = faces[:, 2]
    starts = jnp.concatenate([f0, f1, f2])
    ends = jnp.concatenate([f1, f2, f0])
    p0 = vertices[starts]
    p1 = vertices[ends]
    cen = (p0 + p1) * 0.5
    d = (p1 - p0) + 1e-8
    nc = jnp.sum(cen * cen, axis=1, keepdims=True)
    nd = jnp.sum(d * d, axis=1, keepdims=True)
    rdat = jnp.concatenate([cen, d, nc, nd], axis=1)
    rdat = jnp.pad(rdat, ((0, _EPAD - _E), (0, 0)))
    rdat = rdat.at[_E:, 6].set(1e12)
    cdat = jnp.concatenate(
        [-2.0 * cen, d, nc, nd], axis=1)
    cdat = jnp.pad(cdat, ((0, _EPAD - _E), (0, 0)))
    cdat = cdat.at[_E:, 6].set(1e12)
    cdat = cdat.T

    nsum = pl.pallas_call(
        _pair_kernel,
        grid=(_EPAD // _BI, _EPAD // _BJ),
        in_specs=[
            pl.BlockSpec((_BI, 8), lambda i, j: (i, 0)),
            pl.BlockSpec((8, _BJ), lambda i, j: (0, j)),
        ],
        out_specs=pl.BlockSpec((_BI, 1), lambda i, j: (i, 0)),
        out_shape=jax.ShapeDtypeStruct((_EPAD, 1), jnp.float32),
    )(rdat, cdat)

    n = nsum[:_E, 0]
    n0 = jnp.pad(n[0::3], (0, _FPAD - _F))[None, :]
    n1 = jnp.pad(n[1::3], (0, _FPAD - _F))[None, :]
    n2 = jnp.pad(n[2::3], (0, _FPAD - _F))[None, :]
    fp = jnp.pad(face_probs, (0, _FPAD - _F))[None, :]

    loss = pl.pallas_call(
        _loss_kernel,
        out_shape=jax.ShapeDtypeStruct((1, 1), jnp.float32),
    )(n0, n1, n2, fp)
    return loss[0, 0]
